# baseline (speedup 1.0000x reference)
import functools

import jax
import jax.numpy as jnp
from jax.experimental import pallas as pl
from jax.experimental.pallas import tpu as pltpu

OBS_DIM = 512
ACT_DIM = 32
HID = 1024
LAT = 64
K = 1024
TILE = 2048
SUB = 512


def _row_sq_sum(x):
    s = x * x
    acc = s[:, 0:8]
    for a in range(1, 8):
        acc = acc + s[:, 8 * a:8 * a + 8]
    u = acc[:, 0:4] + acc[:, 4:8]
    u = u[:, 0:2] + u[:, 2:4]
    return u[:, 0:1] + u[:, 1:2]


def _encode(x, we1, be1, we2, be2, we3, be3, cb, csum):
    f32 = jnp.float32
    h = jnp.dot(x, we1, preferred_element_type=f32) + be1
    h = jnp.maximum(h, 0.0)
    h = jnp.dot(h, we2, preferred_element_type=f32) + be2
    h = jnp.maximum(h, 0.0)
    lat = jnp.dot(h, we3, preferred_element_type=f32) + be3
    xsum = _row_sq_sum(lat)
    xc = jax.lax.dot_general(lat, cb, (((1,), (1,)), ((), ())),
                             preferred_element_type=f32)
    d2 = xsum - 2.0 * xc + csum
    return lat, d2


def _vq(lat, d2, cb):
    f32 = jnp.float32
    n = lat.shape[0]
    minval = jnp.min(d2, axis=1, keepdims=True)
    lanes = jax.lax.broadcasted_iota(jnp.int32, (n, K), 1)
    tok = jnp.min(jnp.where(d2 == minval, lanes, K), axis=1)
    onehot = (tok[:, None] == lanes).astype(f32)
    q = jnp.dot(onehot, cb, preferred_element_type=f32)
    qst = lat + (q - lat)
    diff = lat - q
    sq_partial = jnp.sum(diff * diff)
    return tok, qst, sq_partial


def _decode(qst, act, obs, wd1a, wd1b, bd1, wd2, bd2, wd3, bd3):
    f32 = jnp.float32
    bf16 = jnp.bfloat16
    hd = (jnp.dot(qst.astype(bf16), wd1a, preferred_element_type=f32)
          + jnp.dot(act.astype(bf16), wd1b, preferred_element_type=f32)
          + bd1)
    hd = jnp.maximum(hd, 0.0)
    hd = jnp.dot(hd.astype(bf16), wd2, preferred_element_type=f32) + bd2
    hd = jnp.maximum(hd, 0.0)
    rec = jnp.dot(hd.astype(bf16), wd3, preferred_element_type=f32) + bd3
    dr = rec - obs
    rec_partial = jnp.sum(dr * dr)
    return rec, rec_partial


def _fused_kernel(obs_ref, act_ref, we1_ref, be1_ref, we2_ref, be2_ref,
                  we3_ref, be3_ref, cb_ref, csum_ref, wd1_ref, bd1_ref,
                  wd2_ref, bd2_ref, wd3_ref, bd3_ref,
                  recon_ref, tok_ref, qst_ref, lat_ref, part_ref):
    f32 = jnp.float32
    cb = cb_ref[...]
    enc_args = (we1_ref[...], be1_ref[...], we2_ref[...], be2_ref[...],
                we3_ref[...], be3_ref[...], cb, csum_ref[...])
    dec_args = (wd1_ref[0:LAT, :], wd1_ref[LAT:LAT + ACT_DIM, :],
                bd1_ref[...], wd2_ref[...], bd2_ref[...], wd3_ref[...],
                bd3_ref[...])

    nchains = TILE // SUB
    obs_t = [obs_ref[c * SUB:(c + 1) * SUB, :] for c in range(nchains)]
    act_t = [act_ref[c * SUB:(c + 1) * SUB, :] for c in range(nchains)]
    enc_out = [None] * nchains
    sq_total = jnp.float32(0.0)
    rec_total = jnp.float32(0.0)
    enc_out[0] = _encode(jnp.concatenate([obs_t[0], act_t[0]], axis=1),
                         *enc_args)
    for c in range(nchains):
        if c + 1 < nchains:
            enc_out[c + 1] = _encode(
                jnp.concatenate([obs_t[c + 1], act_t[c + 1]], axis=1),
                *enc_args)
        lat, d2 = enc_out[c]
        tok, qst, sq_p = _vq(lat, d2, cb)
        rec, rec_p = _decode(qst, act_t[c], obs_t[c], *dec_args)
        r0 = c * SUB
        lat_ref[r0:r0 + SUB, :] = lat
        tok_ref[0, 0, r0:r0 + SUB] = tok
        qst_ref[r0:r0 + SUB, :] = qst
        recon_ref[r0:r0 + SUB, :] = rec
        sq_total = sq_total + sq_p
        rec_total = rec_total + rec_p

    lane = jax.lax.broadcasted_iota(jnp.int32, (1, 128), 1)
    vec = (jnp.where(lane == 0, sq_total, 0.0)
           + jnp.where(lane == 1, rec_total, 0.0)).astype(f32)
    part_ref[...] = vec.reshape(1, 1, 128)


@functools.partial(jax.jit, static_argnames=())
def kernel(obs, actions, We1, be1, We2, be2, We3, be3, codebook,
           Wd1, bd1, Wd2, bd2, Wd3, bd3):
    b, s = obs.shape[0], obs.shape[1]
    n = b * s
    ntiles = n // TILE
    obs_f = obs.reshape(n, OBS_DIM)
    act_f = actions.reshape(n, ACT_DIM)

    const = lambda i: (0, 0)
    row = lambda i: (i, 0)
    tok_map = lambda i: (i, 0, 0)

    grid_spec = pl.GridSpec(
        grid=(ntiles,),
        in_specs=[
            pl.BlockSpec((TILE, OBS_DIM), row),
            pl.BlockSpec((TILE, ACT_DIM), row),
            pl.BlockSpec(We1.shape, const),
            pl.BlockSpec((1, HID), const),
            pl.BlockSpec(We2.shape, const),
            pl.BlockSpec((1, HID), const),
            pl.BlockSpec(We3.shape, const),
            pl.BlockSpec((1, LAT), const),
            pl.BlockSpec(codebook.shape, const),
            pl.BlockSpec((1, K), const),
            pl.BlockSpec(Wd1.shape, const),
            pl.BlockSpec((1, HID), const),
            pl.BlockSpec(Wd2.shape, const),
            pl.BlockSpec((1, HID), const),
            pl.BlockSpec(Wd3.shape, const),
            pl.BlockSpec((1, OBS_DIM), const),
        ],
        out_specs=[
            pl.BlockSpec((TILE, OBS_DIM), row),
            pl.BlockSpec((1, 1, TILE), tok_map),
            pl.BlockSpec((TILE, LAT), row),
            pl.BlockSpec((TILE, LAT), row),
            pl.BlockSpec((1, 1, 128), tok_map),
        ],
    )

    out_shapes = [
        jax.ShapeDtypeStruct((n, OBS_DIM), jnp.float32),
        jax.ShapeDtypeStruct((ntiles, 1, TILE), jnp.int32),
        jax.ShapeDtypeStruct((n, LAT), jnp.float32),
        jax.ShapeDtypeStruct((n, LAT), jnp.float32),
        jax.ShapeDtypeStruct((ntiles, 1, 128), jnp.float32),
    ]

    recon_f, tok_t, qst_f, lat_f, partials = pl.pallas_call(
        _fused_kernel,
        grid_spec=grid_spec,
        out_shape=out_shapes,
        compiler_params=pltpu.CompilerParams(
            dimension_semantics=("arbitrary",),
        ),
    )(obs_f, act_f, We1, be1.reshape(1, HID), We2, be2.reshape(1, HID),
      We3, be3.reshape(1, LAT), codebook,
      jnp.sum(codebook * codebook, axis=1).reshape(1, K),
      Wd1.astype(jnp.bfloat16), bd1.reshape(1, HID),
      Wd2.astype(jnp.bfloat16), bd2.reshape(1, HID),
      Wd3.astype(jnp.bfloat16), bd3.reshape(1, OBS_DIM))

    reconstructed_obs = recon_f.reshape(b, s, OBS_DIM)
    tokens = tok_t.reshape(b, s)
    quantized_st = qst_f.reshape(b, s, LAT)
    latents = lat_f.reshape(b, s, LAT)

    parts = partials.reshape(ntiles, 128)
    sq_sum = jnp.sum(parts[:, 0])
    rec_sum = jnp.sum(parts[:, 1])
    recon_loss = rec_sum / jnp.float32(n * OBS_DIM)
    codebook_loss = sq_sum / jnp.float32(n * LAT)
    commitment_loss = codebook_loss * jnp.float32(0.25)
    total_quantizer_loss = commitment_loss + codebook_loss
    total_tokenizer_loss = recon_loss + total_quantizer_loss
    return (reconstructed_obs, tokens, quantized_st, latents, recon_loss,
            commitment_loss, codebook_loss, total_quantizer_loss,
            total_tokenizer_loss)


# speedup vs baseline: 1.0225x; 1.0225x over previous
import functools

import jax
import jax.numpy as jnp
from jax.experimental import pallas as pl
from jax.experimental.pallas import tpu as pltpu

OBS_DIM = 512
ACT_DIM = 32
HID = 1024
LAT = 64
K = 1024
TILE = 2048
SUB = 512


def _row_sq_sum(x):
    s = x * x
    acc = s[:, 0:8]
    for a in range(1, 8):
        acc = acc + s[:, 8 * a:8 * a + 8]
    u = acc[:, 0:4] + acc[:, 4:8]
    u = u[:, 0:2] + u[:, 2:4]
    return u[:, 0:1] + u[:, 1:2]


def _encode(x, we1, be1, we2, be2, we3, be3, cb, csum):
    f32 = jnp.float32
    h = jnp.dot(x, we1, preferred_element_type=f32) + be1
    h = jnp.maximum(h, 0.0)
    h = jnp.dot(h, we2, preferred_element_type=f32) + be2
    h = jnp.maximum(h, 0.0)
    lat = jnp.dot(h, we3, preferred_element_type=f32) + be3
    xsum = _row_sq_sum(lat)
    xc = jax.lax.dot_general(lat, cb, (((1,), (1,)), ((), ())),
                             preferred_element_type=f32)
    d2 = xsum - 2.0 * xc + csum
    return lat, d2


def _vq(lat, d2, cb):
    f32 = jnp.float32
    n = lat.shape[0]
    minval = jnp.min(d2, axis=1, keepdims=True)
    lanes = jax.lax.broadcasted_iota(jnp.int32, (n, K), 1)
    tok = jnp.min(jnp.where(d2 == minval, lanes, K), axis=1)
    onehot = (tok[:, None] == lanes).astype(f32)
    q = jnp.dot(onehot, cb, preferred_element_type=f32)
    qst = lat + (q - lat)
    diff = lat - q
    sq_partial = jnp.sum(diff * diff)
    return tok, qst, sq_partial


def _decode(qst, act, obs, wd1a, wd1b, bd1, wd2, bd2, wd3, bd3):
    f32 = jnp.float32
    hd = (jnp.dot(qst, wd1a, preferred_element_type=f32)
          + jnp.dot(act, wd1b, preferred_element_type=f32)
          + bd1)
    hd = jnp.maximum(hd, 0.0)
    hd = jnp.dot(hd, wd2, preferred_element_type=f32) + bd2
    hd = jnp.maximum(hd, 0.0)
    rec = jnp.dot(hd, wd3, preferred_element_type=f32) + bd3
    dr = rec - obs
    rec_partial = jnp.sum(dr * dr)
    return rec, rec_partial


def _fused_kernel(obs_ref, act_ref, we1_ref, be1_ref, we2_ref, be2_ref,
                  we3_ref, be3_ref, cb_ref, csum_ref, wd1_ref, bd1_ref,
                  wd2_ref, bd2_ref, wd3_ref, bd3_ref,
                  recon_ref, tok_ref, qst_ref, lat_ref, part_ref):
    f32 = jnp.float32
    cb = cb_ref[...]
    enc_args = (we1_ref[...], be1_ref[...], we2_ref[...], be2_ref[...],
                we3_ref[...], be3_ref[...], cb, csum_ref[...])
    dec_args = (wd1_ref[0:LAT, :], wd1_ref[LAT:LAT + ACT_DIM, :],
                bd1_ref[...], wd2_ref[...], bd2_ref[...], wd3_ref[...],
                bd3_ref[...])

    nchains = TILE // SUB
    obs_t = [obs_ref[c * SUB:(c + 1) * SUB, :] for c in range(nchains)]
    act_t = [act_ref[c * SUB:(c + 1) * SUB, :] for c in range(nchains)]
    enc_out = [None] * nchains
    sq_total = jnp.float32(0.0)
    rec_total = jnp.float32(0.0)
    enc_out[0] = _encode(jnp.concatenate([obs_t[0], act_t[0]], axis=1),
                         *enc_args)
    for c in range(nchains):
        if c + 1 < nchains:
            enc_out[c + 1] = _encode(
                jnp.concatenate([obs_t[c + 1], act_t[c + 1]], axis=1),
                *enc_args)
        lat, d2 = enc_out[c]
        tok, qst, sq_p = _vq(lat, d2, cb)
        rec, rec_p = _decode(qst, act_t[c], obs_t[c], *dec_args)
        r0 = c * SUB
        lat_ref[r0:r0 + SUB, :] = lat
        tok_ref[0, 0, r0:r0 + SUB] = tok
        qst_ref[r0:r0 + SUB, :] = qst
        recon_ref[r0:r0 + SUB, :] = rec
        sq_total = sq_total + sq_p
        rec_total = rec_total + rec_p

    lane = jax.lax.broadcasted_iota(jnp.int32, (1, 128), 1)
    vec = (jnp.where(lane == 0, sq_total, 0.0)
           + jnp.where(lane == 1, rec_total, 0.0)).astype(f32)
    part_ref[...] = vec.reshape(1, 1, 128)


@functools.partial(jax.jit, static_argnames=())
def kernel(obs, actions, We1, be1, We2, be2, We3, be3, codebook,
           Wd1, bd1, Wd2, bd2, Wd3, bd3):
    b, s = obs.shape[0], obs.shape[1]
    n = b * s
    ntiles = n // TILE
    obs_f = obs.reshape(n, OBS_DIM)
    act_f = actions.reshape(n, ACT_DIM)

    const = lambda i: (0, 0)
    row = lambda i: (i, 0)
    tok_map = lambda i: (i, 0, 0)

    grid_spec = pl.GridSpec(
        grid=(ntiles,),
        in_specs=[
            pl.BlockSpec((TILE, OBS_DIM), row),
            pl.BlockSpec((TILE, ACT_DIM), row),
            pl.BlockSpec(We1.shape, const),
            pl.BlockSpec((1, HID), const),
            pl.BlockSpec(We2.shape, const),
            pl.BlockSpec((1, HID), const),
            pl.BlockSpec(We3.shape, const),
            pl.BlockSpec((1, LAT), const),
            pl.BlockSpec(codebook.shape, const),
            pl.BlockSpec((1, K), const),
            pl.BlockSpec(Wd1.shape, const),
            pl.BlockSpec((1, HID), const),
            pl.BlockSpec(Wd2.shape, const),
            pl.BlockSpec((1, HID), const),
            pl.BlockSpec(Wd3.shape, const),
            pl.BlockSpec((1, OBS_DIM), const),
        ],
        out_specs=[
            pl.BlockSpec((TILE, OBS_DIM), row),
            pl.BlockSpec((1, 1, TILE), tok_map),
            pl.BlockSpec((TILE, LAT), row),
            pl.BlockSpec((TILE, LAT), row),
            pl.BlockSpec((1, 1, 128), tok_map),
        ],
    )

    out_shapes = [
        jax.ShapeDtypeStruct((n, OBS_DIM), jnp.float32),
        jax.ShapeDtypeStruct((ntiles, 1, TILE), jnp.int32),
        jax.ShapeDtypeStruct((n, LAT), jnp.float32),
        jax.ShapeDtypeStruct((n, LAT), jnp.float32),
        jax.ShapeDtypeStruct((ntiles, 1, 128), jnp.float32),
    ]

    recon_f, tok_t, qst_f, lat_f, partials = pl.pallas_call(
        _fused_kernel,
        grid_spec=grid_spec,
        out_shape=out_shapes,
        compiler_params=pltpu.CompilerParams(
            dimension_semantics=("arbitrary",),
        ),
    )(obs_f, act_f, We1, be1.reshape(1, HID), We2, be2.reshape(1, HID),
      We3, be3.reshape(1, LAT), codebook,
      jnp.sum(codebook * codebook, axis=1).reshape(1, K),
      Wd1, bd1.reshape(1, HID), Wd2, bd2.reshape(1, HID),
      Wd3, bd3.reshape(1, OBS_DIM))

    reconstructed_obs = recon_f.reshape(b, s, OBS_DIM)
    tokens = tok_t.reshape(b, s)
    quantized_st = qst_f.reshape(b, s, LAT)
    latents = lat_f.reshape(b, s, LAT)

    parts = partials.reshape(ntiles, 128)
    sq_sum = jnp.sum(parts[:, 0])
    rec_sum = jnp.sum(parts[:, 1])
    recon_loss = rec_sum / jnp.float32(n * OBS_DIM)
    codebook_loss = sq_sum / jnp.float32(n * LAT)
    commitment_loss = codebook_loss * jnp.float32(0.25)
    total_quantizer_loss = commitment_loss + codebook_loss
    total_tokenizer_loss = recon_loss + total_quantizer_loss
    return (reconstructed_obs, tokens, quantized_st, latents, recon_loss,
            commitment_loss, codebook_loss, total_quantizer_loss,
            total_tokenizer_loss)
